# baseline (speedup 1.0000x reference)
import functools

import jax
import jax.numpy as jnp
from jax.experimental import pallas as pl
from jax.experimental.pallas import tpu as pltpu

N_GRAPHS = 16


def _sage_linear_body(a_ref, c_ref, b1_ref, b2_ref, bias_ref, b3_ref, h_ref, pq_ref):
    acc = jnp.dot(a_ref[...], b1_ref[...], preferred_element_type=jnp.float32)
    acc += jnp.dot(c_ref[...], b2_ref[...], preferred_element_type=jnp.float32)
    h = jnp.maximum(acc + bias_ref[...], 0.0)
    h_ref[...] = h
    pq_ref[...] = jnp.dot(h, b3_ref[...], preferred_element_type=jnp.float32)


def _sage_linear(mean, x, WlT, WrT, bl, B3, block_rows=1024):
    Mp, F = mean.shape
    N = WlT.shape[1]
    K3 = B3.shape[1]
    grid = (Mp // block_rows,)
    return pl.pallas_call(
        _sage_linear_body,
        grid=grid,
        in_specs=[
            pl.BlockSpec((block_rows, F), lambda i: (i, 0)),
            pl.BlockSpec((block_rows, F), lambda i: (i, 0)),
            pl.BlockSpec((F, N), lambda i: (0, 0)),
            pl.BlockSpec((F, N), lambda i: (0, 0)),
            pl.BlockSpec((1, N), lambda i: (0, 0)),
            pl.BlockSpec((N, K3), lambda i: (0, 0)),
        ],
        out_specs=[
            pl.BlockSpec((block_rows, N), lambda i: (i, 0)),
            pl.BlockSpec((block_rows, K3), lambda i: (i, 0)),
        ],
        out_shape=[
            jax.ShapeDtypeStruct((Mp, N), jnp.float32),
            jax.ShapeDtypeStruct((Mp, K3), jnp.float32),
        ],
    )(mean, x, WlT, WrT, bl.reshape(1, N), B3)


def _mlp_body(g_ref, a1_ref, b1_ref, a2_ref, b2_ref, out_ref):
    z1 = jnp.maximum(
        jnp.dot(g_ref[...], a1_ref[...], preferred_element_type=jnp.float32)
        + b1_ref[...], 0.0)
    z = jnp.dot(z1, a2_ref[...], preferred_element_type=jnp.float32) + b2_ref[...]
    ncls = 16
    col = jax.lax.broadcasted_iota(jnp.int32, z.shape, 1)
    zm = jnp.where(col < ncls, z, -jnp.inf)
    m = jnp.max(zm, axis=1, keepdims=True)
    ex = jnp.where(col < ncls, jnp.exp(zm - m), 0.0)
    lse = jnp.log(jnp.sum(ex, axis=1, keepdims=True)) + m
    out_ref[...] = z - lse


def _final_mlp(g, Wf1T, bf1, Wf2T, bf2):
    G, H = g.shape
    ncls = Wf2T.shape[1]
    W2p = jnp.zeros((H, 128), jnp.float32).at[:, :ncls].set(Wf2T)
    b2p = jnp.zeros((1, 128), jnp.float32).at[0, :ncls].set(bf2)
    out = pl.pallas_call(
        _mlp_body,
        out_shape=jax.ShapeDtypeStruct((G, 128), jnp.float32),
    )(g, Wf1T, bf1.reshape(1, H), W2p, b2p)
    return out[:, :ncls]


def _seg_softmax(raw, seg, n):
    m = jax.ops.segment_max(raw, seg, num_segments=n)
    ex = jnp.exp(raw - m[seg])
    s = jax.ops.segment_sum(ex, seg, num_segments=n)
    return ex / (s[seg] + 1e-16)


def _match_edges(score, src, dst, node_valid):
    M = node_valid.shape[0]
    E = score.shape[0]
    idx = jnp.arange(E, dtype=jnp.int32)
    NEG = jnp.float32(-jnp.inf)
    BIG = jnp.int32(E)

    def cond(carry):
        rem, taken = carry
        return jnp.any(rem[src] & rem[dst])

    def body(carry):
        rem, taken = carry
        alive = rem[src] & rem[dst]
        sc_a = jnp.where(alive, score, NEG)
        best = jnp.maximum(
            jax.ops.segment_max(sc_a, src, num_segments=M),
            jax.ops.segment_max(sc_a, dst, num_segments=M))
        cand_s = alive & (score == best[src])
        cand_t = alive & (score == best[dst])
        bidx = jnp.minimum(
            jax.ops.segment_min(jnp.where(cand_s, idx, BIG), src, num_segments=M),
            jax.ops.segment_min(jnp.where(cand_t, idx, BIG), dst, num_segments=M))
        dom = cand_s & cand_t & (idx == bidx[src]) & (idx == bidx[dst])
        di = dom.astype(jnp.int32)
        kill = (jax.ops.segment_max(di, src, num_segments=M)
                + jax.ops.segment_max(di, dst, num_segments=M)) > 0
        return rem & ~kill, taken | dom

    rem, taken = jax.lax.while_loop(
        cond, body, (node_valid, jnp.zeros((E,), bool)))

    order = jnp.argsort(jnp.where(taken, -score, jnp.inf), stable=True)
    rank = jnp.zeros((E,), jnp.int32).at[order].set(jnp.arange(E, dtype=jnp.int32))
    n_matched = jnp.sum(taken.astype(jnp.int32))
    cluster = jnp.full((M,), M, dtype=jnp.int32)
    es = jnp.where(taken, src, M)
    et = jnp.where(taken, dst, M)
    cluster = cluster.at[es].set(rank, mode='drop').at[et].set(rank, mode='drop')
    scale = jnp.ones((M + 1,), score.dtype)
    scale = scale.at[jnp.where(taken, rank, M + 1)].set(score, mode='drop')
    tail = n_matched + jnp.cumsum(rem.astype(jnp.int32)) - 1
    cluster = jnp.where(rem, tail, cluster)
    n_new = n_matched + jnp.sum(rem.astype(jnp.int32))
    return cluster, n_new, scale


def _sage_msg(x, src, dst, n):
    msg = jax.ops.segment_sum(x[src], dst, num_segments=n)
    cnt = jax.ops.segment_sum(jnp.ones((src.shape[0],), x.dtype), dst, num_segments=n)
    return msg, cnt


def _pad_rows(a, mp):
    return jnp.zeros((mp,) + a.shape[1:], a.dtype).at[:a.shape[0]].set(a)


def _edge_pool(h, pq, src, dst, batch, node_valid, need_edges):
    M = h.shape[0]
    E = src.shape[0]
    raw = pq[src, 0] + pq[dst, 1]
    score = _seg_softmax(raw, dst, M) + 0.5
    cluster, n_new, scale = _match_edges(score, src, dst, node_valid)
    nx = jax.ops.segment_sum(h, cluster, num_segments=M + 1)
    nx = nx * scale[:, None]
    mx = jax.ops.segment_max(jnp.arange(M), cluster, num_segments=M + 1)
    rows = jnp.arange(M + 1)
    nb = jnp.where(rows < n_new, batch[jnp.clip(mx, 0, M - 1)], N_GRAPHS)
    nv = rows < n_new
    if not need_edges:
        return nx, None, None, nb, nv
    cs = cluster[src]
    cd = cluster[dst]
    key = cs * (M + 1) + cd
    ek = jnp.argsort(key, stable=True)
    ks = key[ek]
    first = jnp.concatenate([jnp.ones((1,), bool), ks[1:] != ks[:-1]])
    keep = first & (ks != M * (M + 1) + M)
    pos = jnp.where(keep, jnp.cumsum(keep) - 1, E)
    new_src = jnp.full((E,), M, jnp.int32).at[pos].set(cs[ek].astype(jnp.int32), mode='drop')
    new_dst = jnp.full((E,), M, jnp.int32).at[pos].set(cd[ek].astype(jnp.int32), mode='drop')
    return nx, new_src, new_dst, nb, nv


def kernel(x, edge_index, batch, W1_l, b1_l, W1_r, W2_l, b2_l, W2_r,
           We1, be1, We2, be2, Wf1, bf1, Wf2, bf2):
    N = x.shape[0]
    PAD = 10240
    src1, dst1 = edge_index[0], edge_index[1]

    msg, cnt = _sage_msg(x, src1, dst1, N)
    mean = msg / jnp.clip(cnt, 1.0)[:, None]
    HID = W1_l.shape[0]
    B3 = jnp.zeros((HID, 128), jnp.float32)
    B3 = B3.at[:, 0].set(We1[0, :HID]).at[:, 1].set(We1[0, HID:])
    h_p, pq_p = _sage_linear(_pad_rows(mean, PAD), _pad_rows(x, PAD),
                             W1_l.T, W1_r.T, b1_l, B3)
    h1 = h_p[:N]
    pq1 = pq_p[:N, :2] + jnp.stack([be1[0], jnp.float32(0.0)])

    nv1 = jnp.ones((N,), bool)
    h2in, src2, dst2, bt2, nv2 = _edge_pool(h1, pq1, src1, dst1, batch, nv1, True)

    M2 = h2in.shape[0]
    msg2, cnt2 = _sage_msg(h2in, src2, dst2, M2)
    mean2 = msg2 / jnp.clip(cnt2, 1.0)[:, None]
    B3b = jnp.zeros((HID, 128), jnp.float32)
    B3b = B3b.at[:, 0].set(We2[0, :HID]).at[:, 1].set(We2[0, HID:])
    h_p2, pq_p2 = _sage_linear(_pad_rows(mean2, PAD), _pad_rows(h2in, PAD),
                               W2_l.T, W2_r.T, b2_l, B3b)
    h2 = h_p2[:M2]
    pq2 = pq_p2[:M2, :2] + jnp.stack([be2[0], jnp.float32(0.0)])

    h3, _, _, bt3, _ = _edge_pool(h2, pq2, src2, dst2, bt2, nv2, False)

    s = jax.ops.segment_sum(h3, bt3, num_segments=N_GRAPHS)
    c = jax.ops.segment_sum(jnp.ones((h3.shape[0],), h3.dtype), bt3,
                            num_segments=N_GRAPHS)
    g = s / jnp.clip(c, 1.0)[:, None]
    return _final_mlp(g, Wf1.T, bf1, Wf2.T, bf2)
